# baseline (speedup 1.0000x reference)
import functools

import jax
import jax.numpy as jnp
from jax import lax
from jax.experimental import pallas as pl
from jax.experimental.pallas import tpu as pltpu
from jax.experimental.pallas import tpu_sc as plsc

N = 10000
E = 320000
D = 128

NC = 2
NS = 16

CHUNK = 120
NCHUNK = 84
EPT = CHUNK * NCHUNK
EPC = EPT * NS
E_PAD = EPC * NC

N_PAD = 10240
RPT = N_PAD // NS


def _final_body(p_ref, w_ref, b_ref, o_ref):
    acc = p_ref[0] + p_ref[1]
    o_ref[...] = jnp.dot(acc, w_ref[...],
                         preferred_element_type=jnp.float32) + b_ref[...]


def _final(partials, w, b):
    return pl.pallas_call(
        _final_body,
        grid=(10,),
        in_specs=[
            pl.BlockSpec((NC, 1000, D), lambda i: (0, i, 0)),
            pl.BlockSpec((D, D), lambda i: (0, 0)),
            pl.BlockSpec((1, D), lambda i: (0, 0)),
        ],
        out_specs=pl.BlockSpec((1000, D), lambda i: (i, 0)),
        out_shape=jax.ShapeDtypeStruct((N, D), jnp.float32),
    )(partials, w, b.reshape(1, D))


NBUF = 3
SB = 3
NGROUP = NCHUNK // SB


def _sc_edge_body(h_hbm, sd_hbm, zeros_hbm, out_hbm,
                  idx, rows, acc_sh, g0, g1, g2, s0, s1, s2):
    gsems = (g0, g1, g2)
    ssems = (s0, s1, s2)
    c = lax.axis_index("c")
    s = lax.axis_index("s")

    pltpu.sync_copy(zeros_hbm, acc_sh.at[pl.ds(s * RPT, RPT)])

    pltpu.sync_copy(sd_hbm.at[c, s, 0], idx.at[0])
    pltpu.sync_copy(sd_hbm.at[c, s, 1], idx.at[1])
    pltpu.async_copy(h_hbm.at[idx.at[0, 0, 0]], rows.at[0], gsems[0])
    pltpu.async_copy(h_hbm.at[idx.at[0, 1, 0]], rows.at[1], gsems[1])

    plsc.subcore_barrier()

    def group(B, _):
        b = lax.rem(B, 2)
        nb = 1 - b
        for j in range(SB):
            k = B * SB + j
            pj = (j - 1) % 3
            pltpu.make_async_copy(
                h_hbm.at[idx.at[b, j, 0]], rows.at[j], gsems[j]).wait()
            pltpu.async_copy(rows.at[j], acc_sh.at[idx.at[b, j, 1]],
                             ssems[j], add=True)

            if j == 0:
                @pl.when(B > 0)
                def _():
                    pltpu.make_async_copy(
                        rows.at[pj], acc_sh.at[idx.at[b, j, 1]],
                        ssems[pj]).wait()

                @pl.when(jnp.logical_and(B > 0, B + 1 < NGROUP))
                def _():
                    pltpu.sync_copy(sd_hbm.at[c, s, B + 1], idx.at[nb])
            else:
                pltpu.make_async_copy(
                    rows.at[pj], acc_sh.at[idx.at[b, j, 1]],
                    ssems[pj]).wait()

            gb = b if j == 0 else nb
            jj = (j + 2) % 3

            @pl.when(k + 2 < NCHUNK)
            def _():
                pltpu.async_copy(h_hbm.at[idx.at[gb, jj, 0]],
                                 rows.at[pj], gsems[pj])
        return ()

    lax.fori_loop(0, NGROUP, group, ())
    pltpu.make_async_copy(rows.at[2], acc_sh.at[idx.at[0, 0, 1]],
                          ssems[2]).wait()
    plsc.subcore_barrier()

    pltpu.sync_copy(acc_sh.at[pl.ds(s * RPT, RPT)],
                    out_hbm.at[c, pl.ds(s * RPT, RPT)])


_sc_edge_kernel = functools.partial(
    pl.kernel,
    out_type=jax.ShapeDtypeStruct((NC, N_PAD, D), jnp.float32),
    mesh=plsc.VectorSubcoreMesh(core_axis_name="c", subcore_axis_name="s"),
    scratch_types=[
        pltpu.VMEM((2, SB, 2, CHUNK), jnp.int32),
        pltpu.VMEM((NBUF, CHUNK, D), jnp.float32),
        pltpu.VMEM_SHARED((N_PAD, D), jnp.float32),
    ] + [pltpu.SemaphoreType.DMA] * 6,
)(_sc_edge_body)


def kernel(input, edge_index, W_l, b_l):
    src = edge_index[0]
    dst = edge_index[1]
    pad = E_PAD - E
    pad_i = jnp.arange(pad, dtype=jnp.int32)
    src_p = jnp.concatenate(
        [src, pad_i % N]).reshape(NC, NS, NGROUP, SB, CHUNK)
    dst_p = jnp.concatenate(
        [dst, N + pad_i % (N_PAD - N)]).reshape(NC, NS, NGROUP, SB, CHUNK)
    sd = jnp.stack([src_p, dst_p], axis=4)

    zeros = jnp.zeros((RPT, D), jnp.float32)
    partials = _sc_edge_kernel(input, sd, zeros)

    return _final(partials, W_l, b_l)


# speedup vs baseline: 1.1510x; 1.1510x over previous
import functools

import jax
import jax.numpy as jnp
from jax import lax
from jax.experimental import pallas as pl
from jax.experimental.pallas import tpu as pltpu
from jax.experimental.pallas import tpu_sc as plsc

N = 10000
E = 320000
D = 128

NC = 2
NS = 16

CHUNK = 128
NCHUNK = 80
EPT = CHUNK * NCHUNK
EPC = EPT * NS
E_PAD = EPC * NC

N_PAD = 10240
RPT = N_PAD // NS


def _final_body(p_ref, w_ref, b_ref, o_ref):
    acc = p_ref[0] + p_ref[1]
    o_ref[...] = jnp.dot(acc, w_ref[...],
                         preferred_element_type=jnp.float32) + b_ref[...]


def _final(partials, w, b):
    return pl.pallas_call(
        _final_body,
        grid=(10,),
        in_specs=[
            pl.BlockSpec((NC, 1000, D), lambda i: (0, i, 0)),
            pl.BlockSpec((D, D), lambda i: (0, 0)),
            pl.BlockSpec((1, D), lambda i: (0, 0)),
        ],
        out_specs=pl.BlockSpec((1000, D), lambda i: (i, 0)),
        out_shape=jax.ShapeDtypeStruct((N, D), jnp.float32),
    )(partials, w, b.reshape(1, D))


NBUF = 2
SB = 4
NGROUP = NCHUNK // SB


def _sc_edge_body(h_hbm, sd_hbm, zeros_hbm, out_hbm,
                  idx, rows, acc_sh, zsem, *sems):
    c = lax.axis_index("c")
    s = lax.axis_index("s")

    zcopy = pltpu.async_copy(zeros_hbm, acc_sh.at[pl.ds(s * RPT, RPT)], zsem)

    pltpu.sync_copy(sd_hbm.at[c, s, 0], idx.at[0])
    pltpu.sync_copy(sd_hbm.at[c, s, 1], idx.at[1])
    for r in range(NBUF):
        pltpu.async_copy(h_hbm.at[idx.at[0, r, 0]], rows.at[r], sems[r])

    zcopy.wait()
    plsc.subcore_barrier()

    def group(B, _):
        b = lax.rem(B, 2)
        nb = lax.rem(B + 1, 2)
        for j in range(SB):
            k = B * SB + j
            r = j % NBUF
            pltpu.make_async_copy(
                h_hbm.at[idx.at[b, j, 0]], rows.at[r], sems[r]).wait()
            pltpu.sync_copy(rows.at[r], acc_sh.at[idx.at[b, j, 1]], add=True)

            gb, gj = (b, j + NBUF) if j + NBUF < SB else (nb, j + NBUF - SB)

            @pl.when(k + NBUF < NCHUNK)
            def _():
                pltpu.async_copy(h_hbm.at[idx.at[gb, gj, 0]],
                                 rows.at[r], sems[r])

        @pl.when(B + 2 < NGROUP)
        def _():
            pltpu.sync_copy(sd_hbm.at[c, s, B + 2], idx.at[b])
        return ()

    lax.fori_loop(0, NGROUP, group, ())
    plsc.subcore_barrier()

    pltpu.sync_copy(acc_sh.at[pl.ds(s * RPT, RPT)],
                    out_hbm.at[c, pl.ds(s * RPT, RPT)])


_sc_edge_kernel = functools.partial(
    pl.kernel,
    out_type=jax.ShapeDtypeStruct((NC, N_PAD, D), jnp.float32),
    mesh=plsc.VectorSubcoreMesh(core_axis_name="c", subcore_axis_name="s"),
    scratch_types=[
        pltpu.VMEM((2, SB, 2, CHUNK), jnp.int32),
        pltpu.VMEM((NBUF, CHUNK, D), jnp.float32),
        pltpu.VMEM_SHARED((N_PAD, D), jnp.float32),
    ] + [pltpu.SemaphoreType.DMA] * (NBUF + 1),
)(_sc_edge_body)


def kernel(input, edge_index, W_l, b_l):
    src = edge_index[0]
    dst = edge_index[1]
    pad = E_PAD - E
    pad_i = jnp.arange(pad, dtype=jnp.int32)
    src_p = jnp.concatenate(
        [src, pad_i % N]).reshape(NC, NS, NGROUP, SB, CHUNK)
    dst_p = jnp.concatenate(
        [dst, N + pad_i % (N_PAD - N)]).reshape(NC, NS, NGROUP, SB, CHUNK)
    sd = jnp.stack([src_p, dst_p], axis=4)

    zeros = jnp.zeros((RPT, D), jnp.float32)
    partials = _sc_edge_kernel(input, sd, zeros)

    return _final(partials, W_l, b_l)
